# baseline (speedup 1.0000x reference)
import functools
import jax
import jax.numpy as jnp
from jax import lax
from jax.experimental import pallas as pl
from jax.experimental.pallas import tpu as pltpu
from jax.experimental.pallas import tpu_sc as plsc

NC, NS, L = 2, 16, 16
NW = NC * NS
R = 16


def kernel(x, pos_table):
    B, S, D = x.shape
    s_per_w = S // NW
    n_chunk = s_per_w // R
    mesh = plsc.VectorSubcoreMesh(core_axis_name="c", subcore_axis_name="s")

    @functools.partial(
        pl.kernel,
        out_type=jax.ShapeDtypeStruct((B, S, D), jnp.float32),
        mesh=mesh,
        scratch_types=[
            pltpu.VMEM((R, D), jnp.float32),
            pltpu.VMEM((R, D), jnp.float32),
        ],
    )
    def k(x_hbm, pos_hbm, out_hbm, pos_v, x_v):
        wid = lax.axis_index("s") * NC + lax.axis_index("c")
        s0 = wid * s_per_w

        def chunk_body(ci, _):
            base = s0 + ci * R
            pltpu.sync_copy(pos_hbm.at[pl.ds(base, R)], pos_v)
            for b in range(B):
                pltpu.sync_copy(x_hbm.at[b, pl.ds(base, R)], x_v)

                def row_body(r, _):
                    for j in range(D // L):
                        sl = pl.ds(j * L, L)
                        x_v[r, sl] = x_v[r, sl] + pos_v[r, sl]
                    return ()

                lax.fori_loop(0, R, row_body, ())
                pltpu.sync_copy(x_v, out_hbm.at[b, pl.ds(base, R)])
            return ()

        lax.fori_loop(0, n_chunk, chunk_body, ())

    return k(x, pos_table)
